# Initial kernel scaffold; baseline (speedup 1.0000x reference)
import jax
import jax.numpy as jnp
from jax.experimental import pallas as pl


def kernel(indices, table):
    raise NotImplementedError("write your pallas kernel here")


# speedup vs baseline: 4.3663x; 4.3663x over previous
import functools

import jax
import jax.numpy as jnp
import numpy as np
from jax import lax
from jax.experimental import pallas as pl
from jax.experimental.pallas import tpu as pltpu
from jax.experimental.pallas import tpu_sc as plsc

_NC = 2
_NS = 16
_L = 16
_NW = _NC * _NS

_DIM = 5
_CHUNK = 6400

_MAPS = np.array(
    [(16 * j + l) // _DIM for j in range(_DIM) for l in range(_L)]
    + [(16 * j + l) % _DIM for j in range(_DIM) for l in range(_L)],
    dtype=np.int32,
)


def _make_body(W, T):
    def _body(idx_hbm, tab_hbm, maps_hbm, out_hbm, idx_v, out_v, tab_v, maps_v):
        wid = lax.axis_index("s") * _NC + lax.axis_index("c")
        pltpu.sync_copy(tab_hbm, tab_v)
        pltpu.sync_copy(maps_hbm, maps_v)
        base = wid * W

        groups = _CHUNK // _L

        for t in range(T):
            cbase = base + t * _CHUNK
            pltpu.sync_copy(idx_hbm.at[pl.ds(cbase, _CHUNK)], idx_v)

            @pl.loop(0, groups)
            def g_body(g):
                cb = g * _L
                ob = g * (_L * _DIM)
                for j in range(_DIM):
                    nj = maps_v[pl.ds(j * _L, _L)]
                    rj = maps_v[pl.ds((_DIM + j) * _L, _L)]
                    iv = plsc.load_gather(idx_v.at[pl.ds(cb, _L)], [nj])
                    gv = iv * _DIM + rj
                    v = plsc.load_gather(tab_v, [gv])
                    out_v[pl.ds(ob + j * _L, _L)] = v

            pltpu.sync_copy(out_v, out_hbm.at[pl.ds(cbase * _DIM, _CHUNK * _DIM)])

    return _body


def kernel(indices, table):
    B, S = indices.shape
    N = B * S
    assert N % (_NW * _CHUNK) == 0
    W = N // _NW
    T = W // _CHUNK

    idx_flat = indices.reshape(-1).astype(jnp.int32)
    tab_flat = jnp.concatenate(
        [table.reshape(-1).astype(jnp.float32), jnp.zeros((28,), jnp.float32)])
    maps = jnp.asarray(_MAPS)

    mesh = plsc.VectorSubcoreMesh(core_axis_name="c", subcore_axis_name="s")
    run = functools.partial(
        pl.kernel,
        mesh=mesh,
        compiler_params=pltpu.CompilerParams(needs_layout_passes=False),
        out_type=jax.ShapeDtypeStruct((N * _DIM,), jnp.float32),
        scratch_types=[
            pltpu.VMEM((_CHUNK,), jnp.int32),
            pltpu.VMEM((_CHUNK * _DIM,), jnp.float32),
            pltpu.VMEM((128,), jnp.float32),
            pltpu.VMEM((160,), jnp.int32),
        ],
    )(_make_body(W, T))
    out_flat = run(idx_flat, tab_flat, maps)
    return out_flat.reshape(B, S, _DIM)


# speedup vs baseline: 4.9370x; 1.1307x over previous
import functools

import jax
import jax.numpy as jnp
import numpy as np
from jax import lax
from jax.experimental import pallas as pl
from jax.experimental.pallas import tpu as pltpu
from jax.experimental.pallas import tpu_sc as plsc

_NC = 2
_NS = 16
_L = 16
_NW = _NC * _NS

_DIM = 5
_CHUNK = 6400

_MAPS = np.array([_DIM * l for l in range(_L)], dtype=np.int32)


def _make_body(W, T):
    def _body(idx_hbm, tab_hbm, maps_hbm, out_hbm, idx_v, out_v, tab_v, maps_v):
        wid = lax.axis_index("s") * _NC + lax.axis_index("c")
        pltpu.sync_copy(tab_hbm, tab_v)
        pltpu.sync_copy(maps_hbm, maps_v)
        base = wid * W

        groups = _CHUNK // _L

        for t in range(T):
            cbase = base + t * _CHUNK
            pltpu.sync_copy(idx_hbm.at[pl.ds(cbase, _CHUNK)], idx_v)

            @pl.loop(0, groups, unroll=8)
            def g_body(g):
                cb = g * _L
                ob = g * (_L * _DIM)
                x5 = idx_v[pl.ds(cb, _L)] * _DIM
                s0 = maps_v[pl.ds(0, _L)]
                owin = out_v.at[pl.ds(ob, _L * _DIM)]
                for c in range(_DIM):
                    v = plsc.load_gather(tab_v, [x5 + c])
                    plsc.store_scatter(owin, [s0 + c], v)

            pltpu.sync_copy(out_v, out_hbm.at[pl.ds(cbase * _DIM, _CHUNK * _DIM)])

    return _body


def kernel(indices, table):
    B, S = indices.shape
    N = B * S
    assert N % (_NW * _CHUNK) == 0
    W = N // _NW
    T = W // _CHUNK

    idx_flat = indices.reshape(-1).astype(jnp.int32)
    tab_flat = jnp.concatenate(
        [table.reshape(-1).astype(jnp.float32), jnp.zeros((28,), jnp.float32)])
    maps = jnp.asarray(_MAPS)

    mesh = plsc.VectorSubcoreMesh(core_axis_name="c", subcore_axis_name="s")
    run = functools.partial(
        pl.kernel,
        mesh=mesh,
        compiler_params=pltpu.CompilerParams(needs_layout_passes=False),
        out_type=jax.ShapeDtypeStruct((N * _DIM,), jnp.float32),
        scratch_types=[
            pltpu.VMEM((_CHUNK,), jnp.int32),
            pltpu.VMEM((_CHUNK * _DIM,), jnp.float32),
            pltpu.VMEM((128,), jnp.float32),
            pltpu.VMEM((_L,), jnp.int32),
        ],
    )(_make_body(W, T))
    out_flat = run(idx_flat, tab_flat, maps)
    return out_flat.reshape(B, S, _DIM)


# speedup vs baseline: 66.4704x; 13.4637x over previous
import functools

import jax
import jax.numpy as jnp
import numpy as np
from jax import lax
from jax.experimental import pallas as pl
from jax.experimental.pallas import tpu as pltpu
from jax.experimental.pallas import tpu_sc as plsc

_NC = 2
_NS = 16
_L = 16
_NW = _NC * _NS

_DIM = 5
_TPAD = 32
_BW = 512
_LR = 8


def _make_body(ntiles, nb):
    ng = _BW // _L

    def _body(idx_hbm, tab_hbm, out_hbm, idx_v, out_v, tab_v):
        wid = lax.axis_index("s") * _NC + lax.axis_index("c")
        pltpu.sync_copy(tab_hbm, tab_v)
        b0 = wid * _BW

        @pl.loop(0, ntiles)
        def chunk(i):
            l0 = i * _LR
            pltpu.sync_copy(idx_hbm.at[pl.ds(l0, _LR), pl.ds(b0, _BW)], idx_v)

            for r in range(_LR):

                @pl.loop(0, ng, unroll=8)
                def g_body(g):
                    x = idx_v[r, pl.ds(g * _L, _L)]
                    for c in range(_DIM):
                        v = plsc.load_gather(tab_v.at[pl.ds(c * _TPAD, _TPAD)], [x])
                        out_v[c, r, pl.ds(g * _L, _L)] = v

            for c in range(_DIM):
                pltpu.sync_copy(
                    out_v.at[c],
                    out_hbm.at[c, pl.ds(l0, _LR), pl.ds(b0, _BW)])

    return _body


def kernel(indices, table):
    B, S = indices.shape
    assert B % (_NW * _BW // _NW) == 0 and S % _LR == 0
    ntiles = S // _LR
    nb = B // _BW

    idx_t = indices.T.astype(jnp.int32)
    tab_t = jnp.pad(table.T.astype(jnp.float32),
                    ((0, 0), (0, _TPAD - table.shape[0]))).reshape(-1)

    mesh = plsc.VectorSubcoreMesh(core_axis_name="c", subcore_axis_name="s")
    run = functools.partial(
        pl.kernel,
        mesh=mesh,
        compiler_params=pltpu.CompilerParams(
            needs_layout_passes=False, use_tc_tiling_on_sc=True),
        out_type=jax.ShapeDtypeStruct((_DIM, S, B), jnp.float32),
        scratch_types=[
            pltpu.VMEM((_LR, _BW), jnp.int32),
            pltpu.VMEM((_DIM, _LR, _BW), jnp.float32),
            pltpu.VMEM((_DIM * _TPAD,), jnp.float32),
        ],
    )(_make_body(ntiles, nb))
    out_t = run(idx_t, tab_t)
    return jnp.transpose(out_t, (2, 1, 0))


# speedup vs baseline: 82.4041x; 1.2397x over previous
import functools

import jax
import jax.numpy as jnp
import numpy as np
from jax import lax
from jax.experimental import pallas as pl
from jax.experimental.pallas import tpu as pltpu
from jax.experimental.pallas import tpu_sc as plsc

_NC = 2
_NS = 16
_L = 16
_NW = _NC * _NS

_DIM = 5
_TPAD = 32
_BW = 512
_LR = 8


def _make_body(ntiles, nb):
    ng = _BW // _L

    def _body(idx_hbm, tab_hbm, out_hbm,
              idx_a, idx_b, out_a, out_b, tab_v,
              in_sem_a, in_sem_b, out_sem_a, out_sem_b):
        wid = lax.axis_index("s") * _NC + lax.axis_index("c")
        pltpu.sync_copy(tab_hbm, tab_v)
        b0 = wid * _BW

        idx_bufs = (idx_a, idx_b)
        out_bufs = (out_a, out_b)
        in_sems = (in_sem_a, in_sem_b)
        out_sems = (out_sem_a, out_sem_b)

        def start_in(i, p):
            return pltpu.async_copy(
                idx_hbm.at[pl.ds(i * _LR, _LR), pl.ds(b0, _BW)],
                idx_bufs[p], in_sems[p])

        def start_outs(i, p):
            return [
                pltpu.async_copy(
                    out_bufs[p].at[c],
                    out_hbm.at[c, pl.ds(i * _LR, _LR), pl.ds(b0, _BW)],
                    out_sems[p])
                for c in range(_DIM)
            ]

        in_h = [start_in(0, 0), start_in(1, 1)]
        out_h = [None, None]

        for i in range(ntiles):
            p = i % 2
            in_h[p].wait()
            if out_h[p] is not None:
                for h in out_h[p]:
                    h.wait()

            idx_v = idx_bufs[p]
            out_v = out_bufs[p]

            @pl.loop(0, _LR)
            def r_body(r):

                @pl.loop(0, ng, unroll=8)
                def g_body(g):
                    x = idx_v[r, pl.ds(g * _L, _L)]
                    for c in range(_DIM):
                        v = plsc.load_gather(tab_v.at[pl.ds(c * _TPAD, _TPAD)], [x])
                        out_v[c, r, pl.ds(g * _L, _L)] = v

            out_h[p] = start_outs(i, p)
            if i + 2 < ntiles:
                in_h[p] = start_in(i + 2, p)

        for hs in out_h:
            for h in hs:
                h.wait()

    return _body


def kernel(indices, table):
    B, S = indices.shape
    assert B % (_NW * _BW // _NW) == 0 and S % _LR == 0
    ntiles = S // _LR
    nb = B // _BW

    idx_t = indices.T.astype(jnp.int32)
    tab_t = jnp.pad(table.T.astype(jnp.float32),
                    ((0, 0), (0, _TPAD - table.shape[0]))).reshape(-1)

    mesh = plsc.VectorSubcoreMesh(core_axis_name="c", subcore_axis_name="s")
    run = functools.partial(
        pl.kernel,
        mesh=mesh,
        compiler_params=pltpu.CompilerParams(
            needs_layout_passes=False, use_tc_tiling_on_sc=True),
        out_type=jax.ShapeDtypeStruct((_DIM, S, B), jnp.float32),
        scratch_types=[
            pltpu.VMEM((_LR, _BW), jnp.int32),
            pltpu.VMEM((_LR, _BW), jnp.int32),
            pltpu.VMEM((_DIM, _LR, _BW), jnp.float32),
            pltpu.VMEM((_DIM, _LR, _BW), jnp.float32),
            pltpu.VMEM((_DIM * _TPAD,), jnp.float32),
            pltpu.SemaphoreType.DMA,
            pltpu.SemaphoreType.DMA,
            pltpu.SemaphoreType.DMA,
            pltpu.SemaphoreType.DMA,
        ],
    )(_make_body(ntiles, nb))
    out_t = run(idx_t, tab_t)
    return jnp.transpose(out_t, (2, 1, 0))
